# baseline (speedup 1.0000x reference)
import functools

import jax
import jax.numpy as jnp
from jax import lax
from jax.experimental import pallas as pl
from jax.experimental.pallas import tpu as pltpu
from jax.experimental.pallas import tpu_sc as plsc

NC = 2
NS = 16
NW = NC * NS
CHUNK = 128
W_TAB = 48
SIZE = 50


def _silu(x):
    return x * jax.nn.sigmoid(x)


def _tc_pre(nodes, coordt, emb_w, emb_b, w_hr, w_hc, b1, n_pad):
    n, _ = nodes.shape
    blk = 1024
    grid = (n_pad // blk,)

    def body(nodes_ref, coordt_ref, embw_ref, embb_ref, whr_ref, whc_ref,
             b1_ref, h_ref, trow_ref, tcol_ref):
        x = nodes_ref[...] @ embw_ref[...] + embb_ref[...]
        h = _silu(x)
        h_ref[...] = h
        c = coordt_ref[...].T
        z = jnp.zeros((h.shape[0], W_TAB - 35), jnp.float32)
        trow_ref[...] = jnp.concatenate(
            [h @ whr_ref[...] + b1_ref[...], c, z], axis=1)
        tcol_ref[...] = jnp.concatenate([h @ whc_ref[...], c, z], axis=1)

    full = lambda a: pl.BlockSpec(a.shape, lambda i: (0,) * a.ndim)
    return pl.pallas_call(
        body,
        grid=grid,
        in_specs=[
            pl.BlockSpec((blk, nodes.shape[1]), lambda i: (i, 0)),
            pl.BlockSpec((3, blk), lambda i: (0, i)),
            full(emb_w), full(emb_b), full(w_hr), full(w_hc), full(b1),
        ],
        out_specs=[
            pl.BlockSpec((blk, 32), lambda i: (i, 0)),
            pl.BlockSpec((blk, W_TAB), lambda i: (i, 0)),
            pl.BlockSpec((blk, W_TAB), lambda i: (i, 0)),
        ],
        out_shape=[
            jax.ShapeDtypeStruct((n_pad, 32), jnp.float32),
            jax.ShapeDtypeStruct((n_pad, W_TAB), jnp.float32),
            jax.ShapeDtypeStruct((n_pad, W_TAB), jnp.float32),
        ],
    )(nodes, coordt, emb_w, emb_b, w_hr, w_hc, b1)


SROWS = 3
SLAB = SROWS * CHUNK


def _sc_gather(trow, tcol, rowi, coli, w_r, e_pad, ns0, ns1):
    srow = SLAB // 4
    mesh = plsc.VectorSubcoreMesh(
        core_axis_name="c", subcore_axis_name="s",
        num_cores=NC, num_subcores=NS)

    @functools.partial(
        pl.kernel,
        out_type=jax.ShapeDtypeStruct((e_pad // 4, 128), jnp.float32),
        mesh=mesh,
        scratch_types=[
            pltpu.VMEM((2, SLAB), jnp.int32),
            pltpu.VMEM((2, SLAB), jnp.int32),
            pltpu.VMEM((2, SLAB, W_TAB), jnp.float32),
            pltpu.VMEM((2, SLAB, W_TAB), jnp.float32),
            pltpu.VMEM((2, srow, 128), jnp.float32),
            pltpu.VMEM((32,), jnp.float32),
            pltpu.SemaphoreType.DMA,
            pltpu.SemaphoreType.DMA,
            pltpu.SemaphoreType.DMA,
        ],
        compiler_params=pltpu.CompilerParams(use_tc_tiling_on_sc=False,
                                             needs_layout_passes=False),
    )
    def gather_k(trow_hbm, tcol_hbm, rowi_hbm, coli_hbm, wr_hbm, g4_hbm,
                 rv, cv, bufa, bufb, gbuf, wrv, sg, sw, si):
        c = lax.axis_index("c")
        s = lax.axis_index("s")
        nsw = lax.select(c == 0, ns0, ns1)
        sbase = lax.select(c == 0, s * ns0, NS * ns0 + s * ns1)
        gbase = sbase * srow
        pltpu.sync_copy(wr_hbm, wrv)
        pltpu.sync_copy(rowi_hbm.at[sbase], rv.at[0])
        pltpu.sync_copy(coli_hbm.at[sbase], cv.at[0])

        def fire_idx(j, p):
            pltpu.async_copy(rowi_hbm.at[sbase + j], rv.at[p], si)
            pltpu.async_copy(coli_hbm.at[sbase + j], cv.at[p], si)

        def drain_idx(j, p):
            pltpu.make_async_copy(rowi_hbm.at[sbase + j], rv.at[p], si).wait()
            pltpu.make_async_copy(coli_hbm.at[sbase + j], cv.at[p], si).wait()

        def fire_gather(j, p):
            pltpu.async_copy(trow_hbm.at[rv.at[p]], bufa.at[p], sg)
            pltpu.async_copy(tcol_hbm.at[cv.at[p]], bufb.at[p], sg)

        def drain_gather(j, p):
            pltpu.make_async_copy(trow_hbm.at[rv.at[p]], bufa.at[p], sg).wait()
            pltpu.make_async_copy(tcol_hbm.at[cv.at[p]], bufb.at[p], sg).wait()

        def fire_write(j, p):
            pltpu.async_copy(
                gbuf.at[p], g4_hbm.at[pl.ds(gbase + j * srow, srow)], sw)

        def drain_write(j, p):
            pltpu.make_async_copy(
                gbuf.at[p], g4_hbm.at[pl.ds(gbase + j * srow, srow)], sw).wait()

        fire_gather(0, 0)

        @pl.when(1 < nsw)
        def _():
            fire_idx(1, 1)

        def body(j, carry):
            p = lax.rem(j, 2)

            @pl.when(j + 1 < nsw)
            def _():
                drain_idx(j + 1, 1 - p)
                fire_gather(j + 1, 1 - p)

            drain_gather(j, p)

            @pl.when(j + 2 < nsw)
            def _():
                fire_idx(j + 2, p)

            @pl.when(j >= 2)
            def _():
                drain_write(j - 2, p)

            w0 = wrv[pl.ds(0, 16)]
            w1 = wrv[pl.ds(16, 16)]

            def combine(i, carry2):
                for u in range(4):
                    e = i * 4 + u
                    d = (bufa[p, e, pl.ds(32, 16)]
                         - bufb[p, e, pl.ds(32, 16)])
                    r = jnp.sum(d * d)
                    g0 = (bufa[p, e, pl.ds(0, 16)]
                          + bufb[p, e, pl.ds(0, 16)] + r * w0)
                    g1 = (bufa[p, e, pl.ds(16, 16)]
                          + bufb[p, e, pl.ds(16, 16)] + r * w1)
                    gbuf[p, i, pl.ds(32 * u, 16)] = g0
                    gbuf[p, i, pl.ds(32 * u + 16, 16)] = g1
                return carry2

            lax.fori_loop(0, srow, combine, 0)
            fire_write(j, p)
            return carry

        lax.fori_loop(0, nsw, body, 0)
        drain_write(nsw - 2, lax.rem(nsw - 2, 2))
        drain_write(nsw - 1, lax.rem(nsw - 1, 2))

    return gather_k(trow, tcol, rowi, coli, w_r)


def _tc_edge(g4, ea4, wea4, e2bd, e2bt, a1bd, a1bt, a2bd, a2bt, sel):
    rows = g4.shape[0]
    blk4 = 512
    grid = (rows // blk4,)

    last_ea_blk = (ea4.shape[0] - 1) // blk4

    def body(g4_ref, ea4_ref, wea4_ref, e2bd_ref, e2bt_ref, a1bd_ref,
             a1bt_ref, a2bd_ref, a2bt_ref, sel_ref, ef_ref):
        x = g4_ref[...] + ea4_ref[...] @ wea4_ref[...]
        m = _silu(x)
        m = _silu(m @ e2bd_ref[...] + e2bt_ref[...])
        t = _silu(m @ a1bd_ref[...] + a1bt_ref[...])
        att = jax.nn.sigmoid(t @ a2bd_ref[...] + a2bt_ref[...])
        ef_ref[...] = m * (att @ sel_ref[...])

    full = lambda a: pl.BlockSpec(a.shape, lambda i: (0,) * a.ndim)
    return pl.pallas_call(
        body,
        grid=grid,
        in_specs=[
            pl.BlockSpec((blk4, 128), lambda i: (i, 0)),
            pl.BlockSpec((blk4, 64), lambda i: (jnp.minimum(i, last_ea_blk), 0)),
            full(wea4), full(e2bd), full(e2bt), full(a1bd), full(a1bt),
            full(a2bd), full(a2bt), full(sel),
        ],
        out_specs=pl.BlockSpec((blk4, 128), lambda i: (i, 0)),
        out_shape=jax.ShapeDtypeStruct((rows, 128), jnp.float32),
    )(g4, ea4, wea4, e2bd, e2bt, a1bd, a1bt, a2bd, a2bt, sel)


def _sc_scatter(ef, rowi_s, zeros_hbm, nacc, cpw):
    epw = cpw * CHUNK
    rps = nacc // NS
    mesh = plsc.VectorSubcoreMesh(
        core_axis_name="c", subcore_axis_name="s",
        num_cores=NC, num_subcores=NS)

    @functools.partial(
        pl.kernel,
        out_type=jax.ShapeDtypeStruct((NC, nacc, 32), jnp.float32),
        mesh=mesh,
        scratch_types=[
            pltpu.VMEM((cpw, CHUNK), jnp.int32),
            pltpu.VMEM((CHUNK, 32), jnp.float32),
            pltpu.VMEM_SHARED((nacc, 32), jnp.float32),
        ],
        compiler_params=pltpu.CompilerParams(use_tc_tiling_on_sc=False),
    )
    def scatter_k(ef_hbm, rowi_hbm, z_hbm, out_hbm, idxv, efv, acc):
        c = lax.axis_index("c")
        s = lax.axis_index("s")
        wid = s * NC + c
        base = wid * epw
        pltpu.sync_copy(z_hbm, acc.at[pl.ds(s * rps, rps)])
        plsc.subcore_barrier()
        pltpu.sync_copy(rowi_hbm.at[wid], idxv)

        def body(j, carry):
            pltpu.sync_copy(ef_hbm.at[pl.ds(base + j * CHUNK, CHUNK)], efv)
            pltpu.sync_copy(efv, acc.at[idxv.at[j]], add=True)
            return carry

        lax.fori_loop(0, cpw, body, 0)
        plsc.subcore_barrier()
        pltpu.sync_copy(acc.at[pl.ds(s * rps, rps)],
                        out_hbm.at[c].at[pl.ds(s * rps, rps)])

    return scatter_k(ef, rowi_s, zeros_hbm)


def _tc_node(h, agg2, node_attr, seg3, n1hw, n1aw, n1nw, n1b, n2w, n2b,
             en1w, en1b, en2w, en2b, n_real):
    n_pad = h.shape[0]
    blk = 1024
    grid = (n_pad // blk,)

    def body(h_ref, agg_ref, na_ref, seg_ref, n1hw_ref, n1aw_ref, n1nw_ref,
             n1b_ref, n2w_ref,
             n2b_ref, en1w_ref, en1b_ref, en2w_ref, en2b_ref, pooled_ref):
        i = pl.program_id(0)
        h = h_ref[...]
        agg = agg_ref[0] + agg_ref[1]
        na_part = lax.dot_general(na_ref[...], n1nw_ref[...],
                                  (((0,), (0,)), ((), ())))
        t = _silu(h @ n1hw_ref[...] + agg @ n1aw_ref[...] + na_part
                  + n1b_ref[...])
        h2 = h + t @ n2w_ref[...] + n2b_ref[...]
        h3 = _silu(h2 @ en1w_ref[...] + en1b_ref[...]) @ en2w_ref[...] + en2b_ref[...]
        ridx = i * blk + lax.broadcasted_iota(jnp.int32, (blk, 1), 0)
        h3 = jnp.where(ridx < n_real, h3, 0.0)
        seg = seg_ref[0, 0, :]
        oh = (lax.broadcasted_iota(jnp.int32, (SIZE, blk), 0)
              == seg[None, :]).astype(jnp.float32)

        @pl.when(i == 0)
        def _():
            pooled_ref[...] = jnp.zeros_like(pooled_ref)

        pooled_ref[...] += oh @ h3

    full = lambda a: pl.BlockSpec(a.shape, lambda i: (0,) * a.ndim)
    return pl.pallas_call(
        body,
        grid=grid,
        in_specs=[
            pl.BlockSpec((blk, 32), lambda i: (i, 0)),
            pl.BlockSpec((2, blk, 32), lambda i: (0, i, 0)),
            pl.BlockSpec((16, blk), lambda i: (0, i)),
            pl.BlockSpec((1, 1, blk), lambda i: (i, 0, 0)),
            full(n1hw), full(n1aw), full(n1nw), full(n1b), full(n2w), full(n2b),
            full(en1w), full(en1b), full(en2w), full(en2b),
        ],
        out_specs=pl.BlockSpec((SIZE, 32), lambda i: (0, 0)),
        out_shape=jax.ShapeDtypeStruct((SIZE, 32), jnp.float32),
    )(h, agg2, node_attr, seg3, n1hw, n1aw, n1nw, n1b, n2w, n2b,
      en1w, en1b, en2w, en2b)


def _tc_decode(pooled, d1w, d1b, d2w, d2b):
    def body(p_ref, d1w_ref, d1b_ref, d2w_ref, d2b_ref, out_ref):
        t = _silu(p_ref[...] @ d1w_ref[...] + d1b_ref[...])
        out_ref[...] = t @ d2w_ref[...] + d2b_ref[...]

    return pl.pallas_call(
        body,
        out_shape=jax.ShapeDtypeStruct((SIZE, 1), jnp.float32),
    )(pooled, d1w, d1b, d2w, d2b)


def kernel(nodes, coord, edges, edge_attr, node_attr, batch, size, params):
    p = params
    n = nodes.shape[0]
    e = edges.shape[1]
    row = edges[0].astype(jnp.int32)
    col = edges[1].astype(jnp.int32)

    nslab = -(-e // (NW * SLAB))
    e_pad = NW * nslab * SLAB
    cpw = e_pad // NW // CHUNK
    pad = e_pad - e
    ns0 = round(2 * nslab * 0.41)
    ns1 = 2 * nslab - ns0

    blkn = 1024
    n_pad = -(-n // blkn) * blkn
    nacc = n_pad

    row_g = jnp.concatenate([row, jnp.zeros((pad,), jnp.int32)])
    col_g = jnp.concatenate([col, jnp.zeros((pad,), jnp.int32)])
    row_s = jnp.concatenate([row, jnp.full((pad,), n, jnp.int32)])
    rowi = row_g.reshape(NW * nslab, SLAB)
    coli = col_g.reshape(NW * nslab, SLAB)
    rowi_s = row_s.reshape(NW, cpw, CHUNK)

    e1w = p['e1_w']
    w_hr = e1w[0:32]
    w_hc = e1w[32:64]
    w_r = e1w[64:65]
    w_ea = e1w[65:81]
    r2 = lambda b: b.reshape(1, -1)

    h, trow, tcol = _tc_pre(nodes, coord.T, p['emb_w'], r2(p['emb_b']),
                            w_hr, w_hc, r2(p['e1_b']), n_pad)
    g4 = _sc_gather(trow, tcol, rowi, coli, w_r.reshape(32), e_pad, ns0, ns1)

    eye4 = jnp.eye(4, dtype=jnp.float32)
    wea4 = jnp.kron(eye4, w_ea)
    ea4 = edge_attr.reshape(e // 4, 64)
    e2bd = jnp.kron(eye4, p['e2_w'])
    e2bt = jnp.tile(p['e2_b'], 4).reshape(1, 128)
    a1bd = jnp.kron(eye4, p['a1_w'])
    a1bt = jnp.tile(p['a1_b'], 4).reshape(1, 128)
    a2bd = jnp.kron(eye4, p['a2_w'])
    a2bt = jnp.tile(p['a2_b'], 4).reshape(1, 4)
    sel = jnp.kron(eye4, jnp.ones((1, 32), jnp.float32))

    ef4 = _tc_edge(g4, ea4, wea4, e2bd, e2bt, a1bd, a1bt,
                   a2bd, a2bt, sel)
    ef = ef4.reshape(e_pad, 32)
    zeros_hbm = jnp.zeros((nacc // NS, 32), jnp.float32)
    agg2 = _sc_scatter(ef, rowi_s, zeros_hbm, nacc, cpw)

    seg = jnp.minimum(batch, size - 1).astype(jnp.int32)
    seg = jnp.concatenate([seg, jnp.full((n_pad - n,), -1, jnp.int32)])
    seg3 = seg.reshape(n_pad // blkn, 1, blkn)

    n1w = p['n1_w']
    pooled = _tc_node(h, agg2, node_attr.T, seg3,
                      n1w[0:32], n1w[32:64], n1w[64:80],
                      r2(p['n1_b']), p['n2_w'], r2(p['n2_b']),
                      p['en1_w'], r2(p['en1_b']), p['en2_w'], r2(p['en2_b']),
                      n)
    return _tc_decode(pooled, p['d1_w'], r2(p['d1_b']),
                      p['d2_w'], r2(p['d2_b']))


# speedup vs baseline: 1.1132x; 1.1132x over previous
import functools

import jax
import jax.numpy as jnp
from jax import lax
from jax.experimental import pallas as pl
from jax.experimental.pallas import tpu as pltpu
from jax.experimental.pallas import tpu_sc as plsc

NC = 2
NS = 16
NW = NC * NS
CHUNK = 128
W_TAB = 48
SIZE = 50


def _silu(x):
    return x * jax.nn.sigmoid(x)


def _tc_pre(nodes, coordt, emb_w, emb_b, w_hr, w_hc, b1, n_pad):
    n, _ = nodes.shape
    blk = 1024
    grid = (n_pad // blk,)

    def body(nodes_ref, coordt_ref, embw_ref, embb_ref, whr_ref, whc_ref,
             b1_ref, h_ref, trow_ref, tcol_ref):
        x = nodes_ref[...] @ embw_ref[...] + embb_ref[...]
        h = _silu(x)
        h_ref[...] = h
        c = coordt_ref[...].T
        z = jnp.zeros((h.shape[0], W_TAB - 35), jnp.float32)
        trow_ref[...] = jnp.concatenate(
            [h @ whr_ref[...] + b1_ref[...], c, z], axis=1)
        tcol_ref[...] = jnp.concatenate([h @ whc_ref[...], c, z], axis=1)

    full = lambda a: pl.BlockSpec(a.shape, lambda i: (0,) * a.ndim)
    return pl.pallas_call(
        body,
        grid=grid,
        in_specs=[
            pl.BlockSpec((blk, nodes.shape[1]), lambda i: (i, 0)),
            pl.BlockSpec((3, blk), lambda i: (0, i)),
            full(emb_w), full(emb_b), full(w_hr), full(w_hc), full(b1),
        ],
        out_specs=[
            pl.BlockSpec((blk, 32), lambda i: (i, 0)),
            pl.BlockSpec((blk, W_TAB), lambda i: (i, 0)),
            pl.BlockSpec((blk, W_TAB), lambda i: (i, 0)),
        ],
        out_shape=[
            jax.ShapeDtypeStruct((n_pad, 32), jnp.float32),
            jax.ShapeDtypeStruct((n_pad, W_TAB), jnp.float32),
            jax.ShapeDtypeStruct((n_pad, W_TAB), jnp.float32),
        ],
    )(nodes, coordt, emb_w, emb_b, w_hr, w_hc, b1)


SROWS = 3
SLAB = SROWS * CHUNK


def _sc_gather(trow, tcol, rowi, coli, w_r, e_pad, ns0, ns1):
    srow = SLAB // 4
    mesh = plsc.VectorSubcoreMesh(
        core_axis_name="c", subcore_axis_name="s",
        num_cores=NC, num_subcores=NS)

    @functools.partial(
        pl.kernel,
        out_type=jax.ShapeDtypeStruct((e_pad // 4, 128), jnp.float32),
        mesh=mesh,
        scratch_types=[
            pltpu.VMEM((2, SLAB), jnp.int32),
            pltpu.VMEM((2, SLAB), jnp.int32),
            pltpu.VMEM((2, SLAB, W_TAB), jnp.float32),
            pltpu.VMEM((2, SLAB, W_TAB), jnp.float32),
            pltpu.VMEM((2, srow, 128), jnp.float32),
            pltpu.VMEM((32,), jnp.float32),
            pltpu.SemaphoreType.DMA,
            pltpu.SemaphoreType.DMA,
            pltpu.SemaphoreType.DMA,
        ],
        compiler_params=pltpu.CompilerParams(use_tc_tiling_on_sc=False,
                                             needs_layout_passes=False),
    )
    def gather_k(trow_hbm, tcol_hbm, rowi_hbm, coli_hbm, wr_hbm, g4_hbm,
                 rv, cv, bufa, bufb, gbuf, wrv, sg, sw, si):
        c = lax.axis_index("c")
        s = lax.axis_index("s")
        nsw = lax.select(c == 0, ns0, ns1)
        sbase = lax.select(c == 0, s * ns0, NS * ns0 + s * ns1)
        gbase = sbase * srow
        pltpu.sync_copy(wr_hbm, wrv)
        pltpu.sync_copy(rowi_hbm.at[sbase], rv.at[0])
        pltpu.sync_copy(coli_hbm.at[sbase], cv.at[0])

        def fire_idx(j, p):
            pltpu.async_copy(rowi_hbm.at[sbase + j], rv.at[p], si)
            pltpu.async_copy(coli_hbm.at[sbase + j], cv.at[p], si)

        def drain_idx(j, p):
            pltpu.make_async_copy(rowi_hbm.at[sbase + j], rv.at[p], si).wait()
            pltpu.make_async_copy(coli_hbm.at[sbase + j], cv.at[p], si).wait()

        def fire_gather(j, p):
            pltpu.async_copy(trow_hbm.at[rv.at[p]], bufa.at[p], sg)
            pltpu.async_copy(tcol_hbm.at[cv.at[p]], bufb.at[p], sg)

        def drain_gather(j, p):
            pltpu.make_async_copy(trow_hbm.at[rv.at[p]], bufa.at[p], sg).wait()
            pltpu.make_async_copy(tcol_hbm.at[cv.at[p]], bufb.at[p], sg).wait()

        def fire_write(j, p):
            pltpu.async_copy(
                gbuf.at[p], g4_hbm.at[pl.ds(gbase + j * srow, srow)], sw)

        def drain_write(j, p):
            pltpu.make_async_copy(
                gbuf.at[p], g4_hbm.at[pl.ds(gbase + j * srow, srow)], sw).wait()

        fire_gather(0, 0)

        @pl.when(1 < nsw)
        def _():
            fire_idx(1, 1)

        def body(j, carry):
            p = lax.rem(j, 2)

            @pl.when(j + 1 < nsw)
            def _():
                drain_idx(j + 1, 1 - p)
                fire_gather(j + 1, 1 - p)

            drain_gather(j, p)

            @pl.when(j + 2 < nsw)
            def _():
                fire_idx(j + 2, p)

            @pl.when(j >= 2)
            def _():
                drain_write(j - 2, p)

            w0 = wrv[pl.ds(0, 16)]
            w1 = wrv[pl.ds(16, 16)]

            def combine(i, carry2):
                for u in range(4):
                    e = i * 4 + u
                    d = (bufa[p, e, pl.ds(32, 16)]
                         - bufb[p, e, pl.ds(32, 16)])
                    r = jnp.sum(d * d)
                    g0 = (bufa[p, e, pl.ds(0, 16)]
                          + bufb[p, e, pl.ds(0, 16)] + r * w0)
                    g1 = (bufa[p, e, pl.ds(16, 16)]
                          + bufb[p, e, pl.ds(16, 16)] + r * w1)
                    gbuf[p, i, pl.ds(32 * u, 16)] = g0
                    gbuf[p, i, pl.ds(32 * u + 16, 16)] = g1
                return carry2

            lax.fori_loop(0, srow, combine, 0)
            fire_write(j, p)
            return carry

        lax.fori_loop(0, nsw, body, 0)
        drain_write(nsw - 2, lax.rem(nsw - 2, 2))
        drain_write(nsw - 1, lax.rem(nsw - 1, 2))

    return gather_k(trow, tcol, rowi, coli, w_r)


def _tc_edge(g4, ea4, wea4, e2bd, e2bt, a1bd, a1bt, a2bd, a2bt, sel):
    rows = g4.shape[0]
    blk4 = 512
    grid = (rows // blk4,)

    last_ea_blk = (ea4.shape[0] - 1) // blk4

    def body(g4_ref, ea4_ref, wea4_ref, e2bd_ref, e2bt_ref, a1bd_ref,
             a1bt_ref, a2bd_ref, a2bt_ref, sel_ref, ef_ref):
        x = g4_ref[...] + ea4_ref[...] @ wea4_ref[...]
        m = _silu(x)
        m = _silu(m @ e2bd_ref[...] + e2bt_ref[...])
        t = _silu(m @ a1bd_ref[...] + a1bt_ref[...])
        att = jax.nn.sigmoid(t @ a2bd_ref[...] + a2bt_ref[...])
        ef_ref[...] = m * (att @ sel_ref[...])

    full = lambda a: pl.BlockSpec(a.shape, lambda i: (0,) * a.ndim)
    return pl.pallas_call(
        body,
        grid=grid,
        in_specs=[
            pl.BlockSpec((blk4, 128), lambda i: (i, 0)),
            pl.BlockSpec((blk4, 64), lambda i: (jnp.minimum(i, last_ea_blk), 0)),
            full(wea4), full(e2bd), full(e2bt), full(a1bd), full(a1bt),
            full(a2bd), full(a2bt), full(sel),
        ],
        out_specs=pl.BlockSpec((blk4, 128), lambda i: (i, 0)),
        out_shape=jax.ShapeDtypeStruct((rows, 128), jnp.float32),
    )(g4, ea4, wea4, e2bd, e2bt, a1bd, a1bt, a2bd, a2bt, sel)


def _sc_scatter(ef, rowi_s, zeros_hbm, nacc, nslab):
    epw = nslab * SLAB
    rps = nacc // NS
    mesh = plsc.VectorSubcoreMesh(
        core_axis_name="c", subcore_axis_name="s",
        num_cores=NC, num_subcores=NS)

    @functools.partial(
        pl.kernel,
        out_type=jax.ShapeDtypeStruct((NC, nacc, 32), jnp.float32),
        mesh=mesh,
        scratch_types=[
            pltpu.VMEM((2, SROWS, CHUNK), jnp.int32),
            pltpu.VMEM((2, SLAB, 32), jnp.float32),
            pltpu.VMEM_SHARED((nacc, 32), jnp.float32),
            pltpu.SemaphoreType.DMA,
            pltpu.SemaphoreType.DMA,
        ],
        compiler_params=pltpu.CompilerParams(use_tc_tiling_on_sc=False),
    )
    def scatter_k(ef_hbm, rowi_hbm, z_hbm, out_hbm, idxv, efv, acc, sl, sa):
        c = lax.axis_index("c")
        s = lax.axis_index("s")
        wid = s * NC + c
        base = wid * epw
        pltpu.sync_copy(z_hbm, acc.at[pl.ds(s * rps, rps)])
        plsc.subcore_barrier()

        def fire_load(j, p):
            pltpu.async_copy(ef_hbm.at[pl.ds(base + j * SLAB, SLAB)],
                             efv.at[p], sl)
            pltpu.async_copy(rowi_hbm.at[wid].at[j], idxv.at[p], sl)

        def drain_load(j, p):
            pltpu.make_async_copy(ef_hbm.at[pl.ds(base + j * SLAB, SLAB)],
                                  efv.at[p], sl).wait()
            pltpu.make_async_copy(rowi_hbm.at[wid].at[j], idxv.at[p],
                                  sl).wait()

        def fire_adds(j, p):
            for q in range(SROWS):
                pltpu.async_copy(efv.at[p].at[pl.ds(q * CHUNK, CHUNK)],
                                 acc.at[idxv.at[p].at[q]], sa, add=True)

        def drain_adds(j, p):
            for q in range(SROWS):
                pltpu.make_async_copy(
                    efv.at[p].at[pl.ds(q * CHUNK, CHUNK)],
                    acc.at[idxv.at[p].at[q]], sa).wait()

        fire_load(0, 0)

        def body(j, carry):
            p = lax.rem(j, 2)
            drain_load(j, p)

            @pl.when(j >= 1)
            def _():
                drain_adds(j - 1, 1 - p)

            @pl.when(j + 1 < nslab)
            def _():
                fire_load(j + 1, 1 - p)

            fire_adds(j, p)
            return carry

        lax.fori_loop(0, nslab, body, 0)
        drain_adds(nslab - 1, lax.rem(nslab - 1, 2))
        plsc.subcore_barrier()
        pltpu.sync_copy(acc.at[pl.ds(s * rps, rps)],
                        out_hbm.at[c].at[pl.ds(s * rps, rps)])

    return scatter_k(ef, rowi_s, zeros_hbm)


def _tc_node(h, agg2, node_attr, seg3, n1hw, n1aw, n1nw, n1b, n2w, n2b,
             en1w, en1b, en2w, en2b, n_real):
    n_pad = h.shape[0]
    blk = 1024
    grid = (n_pad // blk,)

    def body(h_ref, agg_ref, na_ref, seg_ref, n1hw_ref, n1aw_ref, n1nw_ref,
             n1b_ref, n2w_ref,
             n2b_ref, en1w_ref, en1b_ref, en2w_ref, en2b_ref, pooled_ref):
        i = pl.program_id(0)
        h = h_ref[...]
        agg = agg_ref[0] + agg_ref[1]
        na_part = lax.dot_general(na_ref[...], n1nw_ref[...],
                                  (((0,), (0,)), ((), ())))
        t = _silu(h @ n1hw_ref[...] + agg @ n1aw_ref[...] + na_part
                  + n1b_ref[...])
        h2 = h + t @ n2w_ref[...] + n2b_ref[...]
        h3 = _silu(h2 @ en1w_ref[...] + en1b_ref[...]) @ en2w_ref[...] + en2b_ref[...]
        ridx = i * blk + lax.broadcasted_iota(jnp.int32, (blk, 1), 0)
        h3 = jnp.where(ridx < n_real, h3, 0.0)
        seg = seg_ref[0, 0, :]
        oh = (lax.broadcasted_iota(jnp.int32, (SIZE, blk), 0)
              == seg[None, :]).astype(jnp.float32)

        @pl.when(i == 0)
        def _():
            pooled_ref[...] = jnp.zeros_like(pooled_ref)

        pooled_ref[...] += oh @ h3

    full = lambda a: pl.BlockSpec(a.shape, lambda i: (0,) * a.ndim)
    return pl.pallas_call(
        body,
        grid=grid,
        in_specs=[
            pl.BlockSpec((blk, 32), lambda i: (i, 0)),
            pl.BlockSpec((2, blk, 32), lambda i: (0, i, 0)),
            pl.BlockSpec((16, blk), lambda i: (0, i)),
            pl.BlockSpec((1, 1, blk), lambda i: (i, 0, 0)),
            full(n1hw), full(n1aw), full(n1nw), full(n1b), full(n2w), full(n2b),
            full(en1w), full(en1b), full(en2w), full(en2b),
        ],
        out_specs=pl.BlockSpec((SIZE, 32), lambda i: (0, 0)),
        out_shape=jax.ShapeDtypeStruct((SIZE, 32), jnp.float32),
    )(h, agg2, node_attr, seg3, n1hw, n1aw, n1nw, n1b, n2w, n2b,
      en1w, en1b, en2w, en2b)


def _tc_decode(pooled, d1w, d1b, d2w, d2b):
    def body(p_ref, d1w_ref, d1b_ref, d2w_ref, d2b_ref, out_ref):
        t = _silu(p_ref[...] @ d1w_ref[...] + d1b_ref[...])
        out_ref[...] = t @ d2w_ref[...] + d2b_ref[...]

    return pl.pallas_call(
        body,
        out_shape=jax.ShapeDtypeStruct((SIZE, 1), jnp.float32),
    )(pooled, d1w, d1b, d2w, d2b)


def kernel(nodes, coord, edges, edge_attr, node_attr, batch, size, params):
    p = params
    n = nodes.shape[0]
    e = edges.shape[1]
    row = edges[0].astype(jnp.int32)
    col = edges[1].astype(jnp.int32)

    nslab = -(-e // (NW * SLAB))
    e_pad = NW * nslab * SLAB
    cpw = e_pad // NW // CHUNK
    pad = e_pad - e
    ns0 = round(2 * nslab * 0.59)
    ns1 = 2 * nslab - ns0

    blkn = 1024
    n_pad = -(-n // blkn) * blkn
    nacc = n_pad

    row_g = jnp.concatenate([row, jnp.zeros((pad,), jnp.int32)])
    col_g = jnp.concatenate([col, jnp.zeros((pad,), jnp.int32)])
    row_s = jnp.concatenate([row, jnp.full((pad,), n, jnp.int32)])
    rowi = row_g.reshape(NW * nslab, SLAB)
    coli = col_g.reshape(NW * nslab, SLAB)
    rowi_s = row_s.reshape(NW, nslab, SROWS, CHUNK)

    e1w = p['e1_w']
    w_hr = e1w[0:32]
    w_hc = e1w[32:64]
    w_r = e1w[64:65]
    w_ea = e1w[65:81]
    r2 = lambda b: b.reshape(1, -1)

    h, trow, tcol = _tc_pre(nodes, coord.T, p['emb_w'], r2(p['emb_b']),
                            w_hr, w_hc, r2(p['e1_b']), n_pad)
    g4 = _sc_gather(trow, tcol, rowi, coli, w_r.reshape(32), e_pad, ns0, ns1)

    eye4 = jnp.eye(4, dtype=jnp.float32)
    wea4 = jnp.kron(eye4, w_ea)
    ea4 = edge_attr.reshape(e // 4, 64)
    e2bd = jnp.kron(eye4, p['e2_w'])
    e2bt = jnp.tile(p['e2_b'], 4).reshape(1, 128)
    a1bd = jnp.kron(eye4, p['a1_w'])
    a1bt = jnp.tile(p['a1_b'], 4).reshape(1, 128)
    a2bd = jnp.kron(eye4, p['a2_w'])
    a2bt = jnp.tile(p['a2_b'], 4).reshape(1, 4)
    sel = jnp.kron(eye4, jnp.ones((1, 32), jnp.float32))

    ef4 = _tc_edge(g4, ea4, wea4, e2bd, e2bt, a1bd, a1bt,
                   a2bd, a2bt, sel)
    ef = ef4.reshape(e_pad, 32)
    zeros_hbm = jnp.zeros((nacc // NS, 32), jnp.float32)
    agg2 = _sc_scatter(ef, rowi_s, zeros_hbm, nacc, nslab)

    seg = jnp.minimum(batch, size - 1).astype(jnp.int32)
    seg = jnp.concatenate([seg, jnp.full((n_pad - n,), -1, jnp.int32)])
    seg3 = seg.reshape(n_pad // blkn, 1, blkn)

    n1w = p['n1_w']
    pooled = _tc_node(h, agg2, node_attr.T, seg3,
                      n1w[0:32], n1w[32:64], n1w[64:80],
                      r2(p['n1_b']), p['n2_w'], r2(p['n2_b']),
                      p['en1_w'], r2(p['en1_b']), p['en2_w'], r2(p['en2_b']),
                      n)
    return _tc_decode(pooled, p['d1_w'], r2(p['d1_b']),
                      p['d2_w'], r2(p['d2_b']))


# speedup vs baseline: 1.1195x; 1.0057x over previous
import functools

import jax
import jax.numpy as jnp
from jax import lax
from jax.experimental import pallas as pl
from jax.experimental.pallas import tpu as pltpu
from jax.experimental.pallas import tpu_sc as plsc

NC = 2
NS = 16
NW = NC * NS
CHUNK = 128
W_TAB = 48
SIZE = 50


def _silu(x):
    return x * jax.nn.sigmoid(x)


def _tc_pre(nodes, coordt, emb_w, emb_b, w_hr, w_hc, b1, n_pad):
    n, _ = nodes.shape
    blk = 1024
    grid = (n_pad // blk,)

    def body(nodes_ref, coordt_ref, embw_ref, embb_ref, whr_ref, whc_ref,
             b1_ref, h_ref, trow_ref, tcol_ref):
        x = nodes_ref[...] @ embw_ref[...] + embb_ref[...]
        h = _silu(x)
        h_ref[...] = h
        c = coordt_ref[...].T
        z = jnp.zeros((h.shape[0], W_TAB - 35), jnp.float32)
        trow_ref[...] = jnp.concatenate(
            [h @ whr_ref[...] + b1_ref[...], c, z], axis=1)
        tcol_ref[...] = jnp.concatenate([h @ whc_ref[...], c, z], axis=1)

    full = lambda a: pl.BlockSpec(a.shape, lambda i: (0,) * a.ndim)
    return pl.pallas_call(
        body,
        grid=grid,
        in_specs=[
            pl.BlockSpec((blk, nodes.shape[1]), lambda i: (i, 0)),
            pl.BlockSpec((3, blk), lambda i: (0, i)),
            full(emb_w), full(emb_b), full(w_hr), full(w_hc), full(b1),
        ],
        out_specs=[
            pl.BlockSpec((blk, 32), lambda i: (i, 0)),
            pl.BlockSpec((blk, W_TAB), lambda i: (i, 0)),
            pl.BlockSpec((blk, W_TAB), lambda i: (i, 0)),
        ],
        out_shape=[
            jax.ShapeDtypeStruct((n_pad, 32), jnp.float32),
            jax.ShapeDtypeStruct((n_pad, W_TAB), jnp.float32),
            jax.ShapeDtypeStruct((n_pad, W_TAB), jnp.float32),
        ],
    )(nodes, coordt, emb_w, emb_b, w_hr, w_hc, b1)


SROWS = 3
SLAB = SROWS * CHUNK


def _sc_gather(trow, tcol, rowi, coli, w_r, e_pad, ns0, ns1):
    srow = SLAB // 4
    mesh = plsc.VectorSubcoreMesh(
        core_axis_name="c", subcore_axis_name="s",
        num_cores=NC, num_subcores=NS)

    @functools.partial(
        pl.kernel,
        out_type=jax.ShapeDtypeStruct((e_pad // 4, 128), jnp.float32),
        mesh=mesh,
        scratch_types=[
            pltpu.VMEM((2, SLAB), jnp.int32),
            pltpu.VMEM((2, SLAB), jnp.int32),
            pltpu.VMEM((2, SLAB, W_TAB), jnp.float32),
            pltpu.VMEM((2, SLAB, W_TAB), jnp.float32),
            pltpu.VMEM((2, srow, 128), jnp.float32),
            pltpu.VMEM((32,), jnp.float32),
            pltpu.SemaphoreType.DMA,
            pltpu.SemaphoreType.DMA,
            pltpu.SemaphoreType.DMA,
        ],
        compiler_params=pltpu.CompilerParams(use_tc_tiling_on_sc=False,
                                             needs_layout_passes=False),
    )
    def gather_k(trow_hbm, tcol_hbm, rowi_hbm, coli_hbm, wr_hbm, g4_hbm,
                 rv, cv, bufa, bufb, gbuf, wrv, sg, sw, si):
        c = lax.axis_index("c")
        s = lax.axis_index("s")
        nsw = lax.select(c == 0, ns0, ns1)
        sbase = lax.select(c == 0, s * ns0, NS * ns0 + s * ns1)
        gbase = sbase * srow
        pltpu.sync_copy(wr_hbm, wrv)
        pltpu.sync_copy(rowi_hbm.at[sbase], rv.at[0])
        pltpu.sync_copy(coli_hbm.at[sbase], cv.at[0])

        def fire_idx(j, p):
            pltpu.async_copy(rowi_hbm.at[sbase + j], rv.at[p], si)
            pltpu.async_copy(coli_hbm.at[sbase + j], cv.at[p], si)

        def drain_idx(j, p):
            pltpu.make_async_copy(rowi_hbm.at[sbase + j], rv.at[p], si).wait()
            pltpu.make_async_copy(coli_hbm.at[sbase + j], cv.at[p], si).wait()

        def fire_gather(j, p):
            pltpu.async_copy(trow_hbm.at[rv.at[p]], bufa.at[p], sg)
            pltpu.async_copy(tcol_hbm.at[cv.at[p]], bufb.at[p], sg)

        def drain_gather(j, p):
            pltpu.make_async_copy(trow_hbm.at[rv.at[p]], bufa.at[p], sg).wait()
            pltpu.make_async_copy(tcol_hbm.at[cv.at[p]], bufb.at[p], sg).wait()

        def fire_write(j, p):
            pltpu.async_copy(
                gbuf.at[p], g4_hbm.at[pl.ds(gbase + j * srow, srow)], sw)

        def drain_write(j, p):
            pltpu.make_async_copy(
                gbuf.at[p], g4_hbm.at[pl.ds(gbase + j * srow, srow)], sw).wait()

        fire_gather(0, 0)

        @pl.when(1 < nsw)
        def _():
            fire_idx(1, 1)

        def body(j, carry):
            p = lax.rem(j, 2)

            @pl.when(j + 1 < nsw)
            def _():
                drain_idx(j + 1, 1 - p)
                fire_gather(j + 1, 1 - p)

            drain_gather(j, p)

            @pl.when(j + 2 < nsw)
            def _():
                fire_idx(j + 2, p)

            @pl.when(j >= 2)
            def _():
                drain_write(j - 2, p)

            w0 = wrv[pl.ds(0, 16)]
            w1 = wrv[pl.ds(16, 16)]

            def combine(i, carry2):
                for u in range(4):
                    e = i * 4 + u
                    d = (bufa[p, e, pl.ds(32, 16)]
                         - bufb[p, e, pl.ds(32, 16)])
                    r = jnp.sum(d * d)
                    g0 = (bufa[p, e, pl.ds(0, 16)]
                          + bufb[p, e, pl.ds(0, 16)] + r * w0)
                    g1 = (bufa[p, e, pl.ds(16, 16)]
                          + bufb[p, e, pl.ds(16, 16)] + r * w1)
                    gbuf[p, i, pl.ds(32 * u, 16)] = g0
                    gbuf[p, i, pl.ds(32 * u + 16, 16)] = g1
                return carry2

            lax.fori_loop(0, srow, combine, 0)
            fire_write(j, p)
            return carry

        lax.fori_loop(0, nsw, body, 0)
        drain_write(nsw - 2, lax.rem(nsw - 2, 2))
        drain_write(nsw - 1, lax.rem(nsw - 1, 2))

    return gather_k(trow, tcol, rowi, coli, w_r)


def _tc_edge(g4, ea4, wea4, e2bd, e2bt, a1bd, a1bt, a2bd, a2bt, sel):
    rows = g4.shape[0]
    blk4 = 512
    grid = (rows // blk4,)

    last_ea_blk = (ea4.shape[0] - 1) // blk4

    def body(g4_ref, ea4_ref, wea4_ref, e2bd_ref, e2bt_ref, a1bd_ref,
             a1bt_ref, a2bd_ref, a2bt_ref, sel_ref, ef_ref):
        x = g4_ref[...] + ea4_ref[...] @ wea4_ref[...]
        m = _silu(x)
        m = _silu(m @ e2bd_ref[...] + e2bt_ref[...])
        t = _silu(m @ a1bd_ref[...] + a1bt_ref[...])
        att = jax.nn.sigmoid(t @ a2bd_ref[...] + a2bt_ref[...])
        ef_ref[...] = m * (att @ sel_ref[...])

    full = lambda a: pl.BlockSpec(a.shape, lambda i: (0,) * a.ndim)
    return pl.pallas_call(
        body,
        grid=grid,
        in_specs=[
            pl.BlockSpec((blk4, 128), lambda i: (i, 0)),
            pl.BlockSpec((blk4, 64), lambda i: (jnp.minimum(i, last_ea_blk), 0)),
            full(wea4), full(e2bd), full(e2bt), full(a1bd), full(a1bt),
            full(a2bd), full(a2bt), full(sel),
        ],
        out_specs=pl.BlockSpec((blk4, 128), lambda i: (i, 0)),
        out_shape=jax.ShapeDtypeStruct((rows, 128), jnp.float32),
    )(g4, ea4, wea4, e2bd, e2bt, a1bd, a1bt, a2bd, a2bt, sel)


def _sc_scatter(ef, rowi_s, zeros_hbm, nacc, nslab):
    epw = nslab * SLAB
    rps = nacc // NS
    mesh = plsc.VectorSubcoreMesh(
        core_axis_name="c", subcore_axis_name="s",
        num_cores=NC, num_subcores=NS)

    @functools.partial(
        pl.kernel,
        out_type=jax.ShapeDtypeStruct((NC, nacc, 32), jnp.float32),
        mesh=mesh,
        scratch_types=[
            pltpu.VMEM((2, SROWS, CHUNK), jnp.int32),
            pltpu.VMEM((2, SLAB, 32), jnp.float32),
            pltpu.VMEM_SHARED((nacc, 32), jnp.float32),
            pltpu.SemaphoreType.DMA,
            pltpu.SemaphoreType.DMA,
        ],
        compiler_params=pltpu.CompilerParams(use_tc_tiling_on_sc=False),
    )
    def scatter_k(ef_hbm, rowi_hbm, z_hbm, out_hbm, idxv, efv, acc, sl, sa):
        c = lax.axis_index("c")
        s = lax.axis_index("s")
        wid = s * NC + c
        base = wid * epw
        pltpu.sync_copy(z_hbm, acc.at[pl.ds(s * rps, rps)])
        plsc.subcore_barrier()

        def fire_load(j, p):
            pltpu.async_copy(ef_hbm.at[pl.ds(base + j * SLAB, SLAB)],
                             efv.at[p], sl)
            pltpu.async_copy(rowi_hbm.at[wid].at[j], idxv.at[p], sl)

        def drain_load(j, p):
            pltpu.make_async_copy(ef_hbm.at[pl.ds(base + j * SLAB, SLAB)],
                                  efv.at[p], sl).wait()
            pltpu.make_async_copy(rowi_hbm.at[wid].at[j], idxv.at[p],
                                  sl).wait()

        def fire_adds(j, p):
            for q in range(SROWS):
                pltpu.async_copy(efv.at[p].at[pl.ds(q * CHUNK, CHUNK)],
                                 acc.at[idxv.at[p].at[q]], sa, add=True)

        def drain_adds(j, p):
            for q in range(SROWS):
                pltpu.make_async_copy(
                    efv.at[p].at[pl.ds(q * CHUNK, CHUNK)],
                    acc.at[idxv.at[p].at[q]], sa).wait()

        fire_load(0, 0)

        def body(j, carry):
            p = lax.rem(j, 2)
            drain_load(j, p)

            @pl.when(j >= 1)
            def _():
                drain_adds(j - 1, 1 - p)

            @pl.when(j + 1 < nslab)
            def _():
                fire_load(j + 1, 1 - p)

            fire_adds(j, p)
            return carry

        lax.fori_loop(0, nslab, body, 0)
        drain_adds(nslab - 1, lax.rem(nslab - 1, 2))
        plsc.subcore_barrier()
        pltpu.sync_copy(acc.at[pl.ds(s * rps, rps)],
                        out_hbm.at[c].at[pl.ds(s * rps, rps)])

    return scatter_k(ef, rowi_s, zeros_hbm)


def _tc_node(h, agg2, node_attr, seg3, n1hw, n1aw, n1nw, n1b, n2w, n2b,
             en1w, en1b, en2w, en2b, n_real):
    n_pad = h.shape[0]
    blk = 1024
    grid = (n_pad // blk,)

    def body(h_ref, agg_ref, na_ref, seg_ref, n1hw_ref, n1aw_ref, n1nw_ref,
             n1b_ref, n2w_ref,
             n2b_ref, en1w_ref, en1b_ref, en2w_ref, en2b_ref, pooled_ref):
        i = pl.program_id(0)
        h = h_ref[...]
        agg = agg_ref[0] + agg_ref[1]
        na_part = lax.dot_general(na_ref[...], n1nw_ref[...],
                                  (((0,), (0,)), ((), ())))
        t = _silu(h @ n1hw_ref[...] + agg @ n1aw_ref[...] + na_part
                  + n1b_ref[...])
        h2 = h + t @ n2w_ref[...] + n2b_ref[...]
        h3 = _silu(h2 @ en1w_ref[...] + en1b_ref[...]) @ en2w_ref[...] + en2b_ref[...]
        ridx = i * blk + lax.broadcasted_iota(jnp.int32, (blk, 1), 0)
        h3 = jnp.where(ridx < n_real, h3, 0.0)
        seg = seg_ref[0, 0, :]
        oh = (lax.broadcasted_iota(jnp.int32, (SIZE, blk), 0)
              == seg[None, :]).astype(jnp.float32)

        @pl.when(i == 0)
        def _():
            pooled_ref[...] = jnp.zeros_like(pooled_ref)

        pooled_ref[...] += oh @ h3

    full = lambda a: pl.BlockSpec(a.shape, lambda i: (0,) * a.ndim)
    return pl.pallas_call(
        body,
        grid=grid,
        in_specs=[
            pl.BlockSpec((blk, 32), lambda i: (i, 0)),
            pl.BlockSpec((2, blk, 32), lambda i: (0, i, 0)),
            pl.BlockSpec((16, blk), lambda i: (0, i)),
            pl.BlockSpec((1, 1, blk), lambda i: (i, 0, 0)),
            full(n1hw), full(n1aw), full(n1nw), full(n1b), full(n2w), full(n2b),
            full(en1w), full(en1b), full(en2w), full(en2b),
        ],
        out_specs=pl.BlockSpec((SIZE, 32), lambda i: (0, 0)),
        out_shape=jax.ShapeDtypeStruct((SIZE, 32), jnp.float32),
    )(h, agg2, node_attr, seg3, n1hw, n1aw, n1nw, n1b, n2w, n2b,
      en1w, en1b, en2w, en2b)


def _tc_decode(pooled, d1w, d1b, d2w, d2b):
    def body(p_ref, d1w_ref, d1b_ref, d2w_ref, d2b_ref, out_ref):
        t = _silu(p_ref[...] @ d1w_ref[...] + d1b_ref[...])
        out_ref[...] = t @ d2w_ref[...] + d2b_ref[...]

    return pl.pallas_call(
        body,
        out_shape=jax.ShapeDtypeStruct((SIZE, 1), jnp.float32),
    )(pooled, d1w, d1b, d2w, d2b)


def kernel(nodes, coord, edges, edge_attr, node_attr, batch, size, params):
    p = params
    n = nodes.shape[0]
    e = edges.shape[1]
    row = edges[0].astype(jnp.int32)
    col = edges[1].astype(jnp.int32)

    nslab = -(-e // (NW * SLAB))
    e_pad = NW * nslab * SLAB
    cpw = e_pad // NW // CHUNK
    pad = e_pad - e
    ns0 = round(2 * nslab * 0.63)
    ns1 = 2 * nslab - ns0

    blkn = 1024
    n_pad = -(-n // blkn) * blkn
    nacc = n_pad

    row_g = jnp.concatenate([row, jnp.zeros((pad,), jnp.int32)])
    col_g = jnp.concatenate([col, jnp.zeros((pad,), jnp.int32)])
    row_s = jnp.concatenate([row, jnp.full((pad,), n, jnp.int32)])
    rowi = row_g.reshape(NW * nslab, SLAB)
    coli = col_g.reshape(NW * nslab, SLAB)
    rowi_s = row_s.reshape(NW, nslab, SROWS, CHUNK)

    e1w = p['e1_w']
    w_hr = e1w[0:32]
    w_hc = e1w[32:64]
    w_r = e1w[64:65]
    w_ea = e1w[65:81]
    r2 = lambda b: b.reshape(1, -1)

    h, trow, tcol = _tc_pre(nodes, coord.T, p['emb_w'], r2(p['emb_b']),
                            w_hr, w_hc, r2(p['e1_b']), n_pad)
    g4 = _sc_gather(trow, tcol, rowi, coli, w_r.reshape(32), e_pad, ns0, ns1)

    eye4 = jnp.eye(4, dtype=jnp.float32)
    wea4 = jnp.kron(eye4, w_ea)
    ea4 = edge_attr.reshape(e // 4, 64)
    e2bd = jnp.kron(eye4, p['e2_w'])
    e2bt = jnp.tile(p['e2_b'], 4).reshape(1, 128)
    a1bd = jnp.kron(eye4, p['a1_w'])
    a1bt = jnp.tile(p['a1_b'], 4).reshape(1, 128)
    a2bd = jnp.kron(eye4, p['a2_w'])
    a2bt = jnp.tile(p['a2_b'], 4).reshape(1, 4)
    sel = jnp.kron(eye4, jnp.ones((1, 32), jnp.float32))

    ef4 = _tc_edge(g4, ea4, wea4, e2bd, e2bt, a1bd, a1bt,
                   a2bd, a2bt, sel)
    ef = ef4.reshape(e_pad, 32)
    zeros_hbm = jnp.zeros((nacc // NS, 32), jnp.float32)
    agg2 = _sc_scatter(ef, rowi_s, zeros_hbm, nacc, nslab)

    seg = jnp.minimum(batch, size - 1).astype(jnp.int32)
    seg = jnp.concatenate([seg, jnp.full((n_pad - n,), -1, jnp.int32)])
    seg3 = seg.reshape(n_pad // blkn, 1, blkn)

    n1w = p['n1_w']
    pooled = _tc_node(h, agg2, node_attr.T, seg3,
                      n1w[0:32], n1w[32:64], n1w[64:80],
                      r2(p['n1_b']), p['n2_w'], r2(p['n2_b']),
                      p['en1_w'], r2(p['en1_b']), p['en2_w'], r2(p['en2_b']),
                      n)
    return _tc_decode(pooled, p['d1_w'], r2(p['d1_b']),
                      p['d2_w'], r2(p['d2_b']))


# speedup vs baseline: 1.1257x; 1.0055x over previous
import functools

import jax
import jax.numpy as jnp
from jax import lax
from jax.experimental import pallas as pl
from jax.experimental.pallas import tpu as pltpu
from jax.experimental.pallas import tpu_sc as plsc

NC = 2
NS = 16
NW = NC * NS
CHUNK = 128
W_TAB = 48
SIZE = 50


def _silu(x):
    return x * jax.nn.sigmoid(x)


def _tc_pre(nodes, coordt, emb_w, emb_b, w_hr, w_hc, b1, n_pad):
    n, _ = nodes.shape
    blk = 1024
    grid = (n_pad // blk,)

    def body(nodes_ref, coordt_ref, embw_ref, embb_ref, whr_ref, whc_ref,
             b1_ref, h_ref, trow_ref, tcol_ref):
        x = nodes_ref[...] @ embw_ref[...] + embb_ref[...]
        h = _silu(x)
        h_ref[...] = h
        c = coordt_ref[...].T
        z = jnp.zeros((h.shape[0], W_TAB - 35), jnp.float32)
        trow_ref[...] = jnp.concatenate(
            [h @ whr_ref[...] + b1_ref[...], c, z], axis=1)
        tcol_ref[...] = jnp.concatenate([h @ whc_ref[...], c, z], axis=1)

    full = lambda a: pl.BlockSpec(a.shape, lambda i: (0,) * a.ndim)
    return pl.pallas_call(
        body,
        grid=grid,
        in_specs=[
            pl.BlockSpec((blk, nodes.shape[1]), lambda i: (i, 0)),
            pl.BlockSpec((3, blk), lambda i: (0, i)),
            full(emb_w), full(emb_b), full(w_hr), full(w_hc), full(b1),
        ],
        out_specs=[
            pl.BlockSpec((blk, 32), lambda i: (i, 0)),
            pl.BlockSpec((blk, W_TAB), lambda i: (i, 0)),
            pl.BlockSpec((blk, W_TAB), lambda i: (i, 0)),
        ],
        out_shape=[
            jax.ShapeDtypeStruct((n_pad, 32), jnp.float32),
            jax.ShapeDtypeStruct((n_pad, W_TAB), jnp.float32),
            jax.ShapeDtypeStruct((n_pad, W_TAB), jnp.float32),
        ],
    )(nodes, coordt, emb_w, emb_b, w_hr, w_hc, b1)


SROWS = 3
SLAB = SROWS * CHUNK


def _sc_gather(trow, tcol, rowi, coli, w_r, e_pad, ns0, ns1):
    srow = SLAB // 4
    mesh = plsc.VectorSubcoreMesh(
        core_axis_name="c", subcore_axis_name="s",
        num_cores=NC, num_subcores=NS)

    @functools.partial(
        pl.kernel,
        out_type=jax.ShapeDtypeStruct((e_pad // 4, 128), jnp.float32),
        mesh=mesh,
        scratch_types=[
            pltpu.VMEM((2, SLAB), jnp.int32),
            pltpu.VMEM((2, SLAB), jnp.int32),
            pltpu.VMEM((2, SLAB, W_TAB), jnp.float32),
            pltpu.VMEM((2, SLAB, W_TAB), jnp.float32),
            pltpu.VMEM((2, srow, 128), jnp.float32),
            pltpu.VMEM((32,), jnp.float32),
            pltpu.SemaphoreType.DMA,
            pltpu.SemaphoreType.DMA,
            pltpu.SemaphoreType.DMA,
        ],
        compiler_params=pltpu.CompilerParams(use_tc_tiling_on_sc=False,
                                             needs_layout_passes=False),
    )
    def gather_k(trow_hbm, tcol_hbm, rowi_hbm, coli_hbm, wr_hbm, g4_hbm,
                 rv, cv, bufa, bufb, gbuf, wrv, sg, sw, si):
        c = lax.axis_index("c")
        s = lax.axis_index("s")
        nsw = lax.select(c == 0, ns0, ns1)
        sbase = lax.select(c == 0, s * ns0, NS * ns0 + s * ns1)
        gbase = sbase * srow
        pltpu.sync_copy(wr_hbm, wrv)
        pltpu.sync_copy(rowi_hbm.at[sbase], rv.at[0])
        pltpu.sync_copy(coli_hbm.at[sbase], cv.at[0])

        def fire_idx(j, p):
            pltpu.async_copy(rowi_hbm.at[sbase + j], rv.at[p], si)
            pltpu.async_copy(coli_hbm.at[sbase + j], cv.at[p], si)

        def drain_idx(j, p):
            pltpu.make_async_copy(rowi_hbm.at[sbase + j], rv.at[p], si).wait()
            pltpu.make_async_copy(coli_hbm.at[sbase + j], cv.at[p], si).wait()

        def fire_gather(j, p):
            pltpu.async_copy(trow_hbm.at[rv.at[p]], bufa.at[p], sg)
            pltpu.async_copy(tcol_hbm.at[cv.at[p]], bufb.at[p], sg)

        def drain_gather(j, p):
            pltpu.make_async_copy(trow_hbm.at[rv.at[p]], bufa.at[p], sg).wait()
            pltpu.make_async_copy(tcol_hbm.at[cv.at[p]], bufb.at[p], sg).wait()

        def fire_write(j, p):
            pltpu.async_copy(
                gbuf.at[p], g4_hbm.at[pl.ds(gbase + j * srow, srow)], sw)

        def drain_write(j, p):
            pltpu.make_async_copy(
                gbuf.at[p], g4_hbm.at[pl.ds(gbase + j * srow, srow)], sw).wait()

        fire_gather(0, 0)

        @pl.when(1 < nsw)
        def _():
            fire_idx(1, 1)

        def body(j, carry):
            p = lax.rem(j, 2)

            @pl.when(j + 1 < nsw)
            def _():
                drain_idx(j + 1, 1 - p)
                fire_gather(j + 1, 1 - p)

            drain_gather(j, p)

            @pl.when(j + 2 < nsw)
            def _():
                fire_idx(j + 2, p)

            @pl.when(j >= 2)
            def _():
                drain_write(j - 2, p)

            w0 = wrv[pl.ds(0, 16)]
            w1 = wrv[pl.ds(16, 16)]

            def combine(i, carry2):
                for u in range(4):
                    e = i * 4 + u
                    d = (bufa[p, e, pl.ds(32, 16)]
                         - bufb[p, e, pl.ds(32, 16)])
                    r = jnp.sum(d * d)
                    g0 = (bufa[p, e, pl.ds(0, 16)]
                          + bufb[p, e, pl.ds(0, 16)] + r * w0)
                    g1 = (bufa[p, e, pl.ds(16, 16)]
                          + bufb[p, e, pl.ds(16, 16)] + r * w1)
                    gbuf[p, i, pl.ds(32 * u, 16)] = g0
                    gbuf[p, i, pl.ds(32 * u + 16, 16)] = g1
                return carry2

            lax.fori_loop(0, srow, combine, 0)
            fire_write(j, p)
            return carry

        lax.fori_loop(0, nsw, body, 0)
        drain_write(nsw - 2, lax.rem(nsw - 2, 2))
        drain_write(nsw - 1, lax.rem(nsw - 1, 2))

    return gather_k(trow, tcol, rowi, coli, w_r)


def _tc_edge(g4, ea4, wea4, e2bd, e2bt, a1bd, a1bt, a2bd, a2bt, sel):
    rows = g4.shape[0]
    blk4 = 512
    grid = (rows // blk4,)

    last_ea_blk = (ea4.shape[0] - 1) // blk4

    def body(g4_ref, ea4_ref, wea4_ref, e2bd_ref, e2bt_ref, a1bd_ref,
             a1bt_ref, a2bd_ref, a2bt_ref, sel_ref, ef_ref):
        x = g4_ref[...] + ea4_ref[...] @ wea4_ref[...]
        m = _silu(x)
        m = _silu(m @ e2bd_ref[...] + e2bt_ref[...])
        t = _silu(m @ a1bd_ref[...] + a1bt_ref[...])
        att = jax.nn.sigmoid(t @ a2bd_ref[...] + a2bt_ref[...])
        ef_ref[...] = m * (att @ sel_ref[...])

    full = lambda a: pl.BlockSpec(a.shape, lambda i: (0,) * a.ndim)
    return pl.pallas_call(
        body,
        grid=grid,
        in_specs=[
            pl.BlockSpec((blk4, 128), lambda i: (i, 0)),
            pl.BlockSpec((blk4, 64), lambda i: (jnp.minimum(i, last_ea_blk), 0)),
            full(wea4), full(e2bd), full(e2bt), full(a1bd), full(a1bt),
            full(a2bd), full(a2bt), full(sel),
        ],
        out_specs=pl.BlockSpec((blk4, 128), lambda i: (i, 0)),
        out_shape=jax.ShapeDtypeStruct((rows, 128), jnp.float32),
    )(g4, ea4, wea4, e2bd, e2bt, a1bd, a1bt, a2bd, a2bt, sel)


def _sc_scatter(ef, rowi_s, zeros_hbm, nacc, nslab):
    epw = nslab * SLAB
    rps = nacc // NS
    mesh = plsc.VectorSubcoreMesh(
        core_axis_name="c", subcore_axis_name="s",
        num_cores=NC, num_subcores=NS)

    @functools.partial(
        pl.kernel,
        out_type=jax.ShapeDtypeStruct((NC, nacc, 32), jnp.float32),
        mesh=mesh,
        scratch_types=[
            pltpu.VMEM((2, SROWS, CHUNK), jnp.int32),
            pltpu.VMEM((2, SLAB, 32), jnp.float32),
            pltpu.VMEM_SHARED((nacc, 32), jnp.float32),
            pltpu.SemaphoreType.DMA,
            pltpu.SemaphoreType.DMA,
        ],
        compiler_params=pltpu.CompilerParams(use_tc_tiling_on_sc=False),
    )
    def scatter_k(ef_hbm, rowi_hbm, z_hbm, out_hbm, idxv, efv, acc, sl, sa):
        c = lax.axis_index("c")
        s = lax.axis_index("s")
        wid = s * NC + c
        base = wid * epw
        pltpu.sync_copy(z_hbm, acc.at[pl.ds(s * rps, rps)])
        plsc.subcore_barrier()

        def fire_load(j, p):
            pltpu.async_copy(ef_hbm.at[pl.ds(base + j * SLAB, SLAB)],
                             efv.at[p], sl)
            pltpu.async_copy(rowi_hbm.at[wid].at[j], idxv.at[p], sl)

        def drain_load(j, p):
            pltpu.make_async_copy(ef_hbm.at[pl.ds(base + j * SLAB, SLAB)],
                                  efv.at[p], sl).wait()
            pltpu.make_async_copy(rowi_hbm.at[wid].at[j], idxv.at[p],
                                  sl).wait()

        def fire_adds(j, p):
            for q in range(SROWS):
                pltpu.async_copy(efv.at[p].at[pl.ds(q * CHUNK, CHUNK)],
                                 acc.at[idxv.at[p].at[q]], sa, add=True)

        def drain_adds(j, p):
            for q in range(SROWS):
                pltpu.make_async_copy(
                    efv.at[p].at[pl.ds(q * CHUNK, CHUNK)],
                    acc.at[idxv.at[p].at[q]], sa).wait()

        fire_load(0, 0)

        def body(j, carry):
            p = lax.rem(j, 2)
            drain_load(j, p)

            @pl.when(j >= 1)
            def _():
                drain_adds(j - 1, 1 - p)

            @pl.when(j + 1 < nslab)
            def _():
                fire_load(j + 1, 1 - p)

            fire_adds(j, p)
            return carry

        lax.fori_loop(0, nslab, body, 0)
        drain_adds(nslab - 1, lax.rem(nslab - 1, 2))
        plsc.subcore_barrier()
        pltpu.sync_copy(acc.at[pl.ds(s * rps, rps)],
                        out_hbm.at[c].at[pl.ds(s * rps, rps)])

    return scatter_k(ef, rowi_s, zeros_hbm)


def _tc_node(h, agg2, node_attr, seg3, n1hw, n1aw, n1nw, n1b, n2w, n2b,
             en1w, en1b, en2w, en2b, n_real):
    n_pad = h.shape[0]
    blk = 1024
    grid = (n_pad // blk,)

    def body(h_ref, agg_ref, na_ref, seg_ref, n1hw_ref, n1aw_ref, n1nw_ref,
             n1b_ref, n2w_ref,
             n2b_ref, en1w_ref, en1b_ref, en2w_ref, en2b_ref, pooled_ref):
        i = pl.program_id(0)
        h = h_ref[...]
        agg = agg_ref[0] + agg_ref[1]
        na_part = lax.dot_general(na_ref[...], n1nw_ref[...],
                                  (((0,), (0,)), ((), ())))
        t = _silu(h @ n1hw_ref[...] + agg @ n1aw_ref[...] + na_part
                  + n1b_ref[...])
        h2 = h + t @ n2w_ref[...] + n2b_ref[...]
        h3 = _silu(h2 @ en1w_ref[...] + en1b_ref[...]) @ en2w_ref[...] + en2b_ref[...]
        ridx = i * blk + lax.broadcasted_iota(jnp.int32, (blk, 1), 0)
        h3 = jnp.where(ridx < n_real, h3, 0.0)
        seg = seg_ref[0, 0, :]
        oh = (lax.broadcasted_iota(jnp.int32, (SIZE, blk), 0)
              == seg[None, :]).astype(jnp.float32)

        @pl.when(i == 0)
        def _():
            pooled_ref[...] = jnp.zeros_like(pooled_ref)

        pooled_ref[...] += oh @ h3

    full = lambda a: pl.BlockSpec(a.shape, lambda i: (0,) * a.ndim)
    return pl.pallas_call(
        body,
        grid=grid,
        in_specs=[
            pl.BlockSpec((blk, 32), lambda i: (i, 0)),
            pl.BlockSpec((2, blk, 32), lambda i: (0, i, 0)),
            pl.BlockSpec((16, blk), lambda i: (0, i)),
            pl.BlockSpec((1, 1, blk), lambda i: (i, 0, 0)),
            full(n1hw), full(n1aw), full(n1nw), full(n1b), full(n2w), full(n2b),
            full(en1w), full(en1b), full(en2w), full(en2b),
        ],
        out_specs=pl.BlockSpec((SIZE, 32), lambda i: (0, 0)),
        out_shape=jax.ShapeDtypeStruct((SIZE, 32), jnp.float32),
    )(h, agg2, node_attr, seg3, n1hw, n1aw, n1nw, n1b, n2w, n2b,
      en1w, en1b, en2w, en2b)


def _tc_decode(pooled, d1w, d1b, d2w, d2b):
    def body(p_ref, d1w_ref, d1b_ref, d2w_ref, d2b_ref, out_ref):
        t = _silu(p_ref[...] @ d1w_ref[...] + d1b_ref[...])
        out_ref[...] = t @ d2w_ref[...] + d2b_ref[...]

    return pl.pallas_call(
        body,
        out_shape=jax.ShapeDtypeStruct((SIZE, 1), jnp.float32),
    )(pooled, d1w, d1b, d2w, d2b)


def kernel(nodes, coord, edges, edge_attr, node_attr, batch, size, params):
    p = params
    n = nodes.shape[0]
    e = edges.shape[1]
    row = edges[0].astype(jnp.int32)
    col = edges[1].astype(jnp.int32)

    nslab = -(-e // (NW * SLAB))
    e_pad = NW * nslab * SLAB
    cpw = e_pad // NW // CHUNK
    pad = e_pad - e
    ns0 = round(2 * nslab * 0.67)
    ns1 = 2 * nslab - ns0

    blkn = 1024
    n_pad = -(-n // blkn) * blkn
    nacc = n_pad

    row_g = jnp.concatenate([row, jnp.zeros((pad,), jnp.int32)])
    col_g = jnp.concatenate([col, jnp.zeros((pad,), jnp.int32)])
    row_s = jnp.concatenate([row, jnp.full((pad,), n, jnp.int32)])
    rowi = row_g.reshape(NW * nslab, SLAB)
    coli = col_g.reshape(NW * nslab, SLAB)
    rowi_s = row_s.reshape(NW, nslab, SROWS, CHUNK)

    e1w = p['e1_w']
    w_hr = e1w[0:32]
    w_hc = e1w[32:64]
    w_r = e1w[64:65]
    w_ea = e1w[65:81]
    r2 = lambda b: b.reshape(1, -1)

    h, trow, tcol = _tc_pre(nodes, coord.T, p['emb_w'], r2(p['emb_b']),
                            w_hr, w_hc, r2(p['e1_b']), n_pad)
    g4 = _sc_gather(trow, tcol, rowi, coli, w_r.reshape(32), e_pad, ns0, ns1)

    eye4 = jnp.eye(4, dtype=jnp.float32)
    wea4 = jnp.kron(eye4, w_ea)
    ea4 = edge_attr.reshape(e // 4, 64)
    e2bd = jnp.kron(eye4, p['e2_w'])
    e2bt = jnp.tile(p['e2_b'], 4).reshape(1, 128)
    a1bd = jnp.kron(eye4, p['a1_w'])
    a1bt = jnp.tile(p['a1_b'], 4).reshape(1, 128)
    a2bd = jnp.kron(eye4, p['a2_w'])
    a2bt = jnp.tile(p['a2_b'], 4).reshape(1, 4)
    sel = jnp.kron(eye4, jnp.ones((1, 32), jnp.float32))

    ef4 = _tc_edge(g4, ea4, wea4, e2bd, e2bt, a1bd, a1bt,
                   a2bd, a2bt, sel)
    ef = ef4.reshape(e_pad, 32)
    zeros_hbm = jnp.zeros((nacc // NS, 32), jnp.float32)
    agg2 = _sc_scatter(ef, rowi_s, zeros_hbm, nacc, nslab)

    seg = jnp.minimum(batch, size - 1).astype(jnp.int32)
    seg = jnp.concatenate([seg, jnp.full((n_pad - n,), -1, jnp.int32)])
    seg3 = seg.reshape(n_pad // blkn, 1, blkn)

    n1w = p['n1_w']
    pooled = _tc_node(h, agg2, node_attr.T, seg3,
                      n1w[0:32], n1w[32:64], n1w[64:80],
                      r2(p['n1_b']), p['n2_w'], r2(p['n2_b']),
                      p['en1_w'], r2(p['en1_b']), p['en2_w'], r2(p['en2_b']),
                      n)
    return _tc_decode(pooled, p['d1_w'], r2(p['d1_b']),
                      p['d2_w'], r2(p['d2_b']))
